# baseline (speedup 1.0000x reference)
import jax
import jax.numpy as jnp
from jax.experimental import pallas as pl

_NB_KNN = (10, 20, 100, 200)
_MAXK = 200
_TEMP = 0.07
_NCLS = 1000


def _simmat_kernel(x_ref, t_ref, s_ref):
    s_ref[...] = jax.lax.dot_general(
        x_ref[...], t_ref[...], (((1,), (1,)), ((), ())),
        preferred_element_type=jnp.float32,
        precision=jax.lax.Precision.DEFAULT)


def _similarity(features_rank, train_features):
    q, d = features_rank.shape
    n = train_features.shape[0]
    bq, bn = 256, 512
    return pl.pallas_call(
        _simmat_kernel,
        grid=(q // bq, pl.cdiv(n, bn)),
        in_specs=[pl.BlockSpec((bq, d), lambda i, j: (i, 0)),
                  pl.BlockSpec((bn, d), lambda i, j: (j, 0))],
        out_specs=pl.BlockSpec((bq, bn), lambda i, j: (i, j)),
        out_shape=jax.ShapeDtypeStruct((q, n), jnp.float32),
    )(features_rank, train_features)


def kernel(features_rank, train_features, train_labels):
    sims = _similarity(features_rank, train_features)
    topk_sims, indices = jax.lax.top_k(sims, _MAXK)
    neighbors_labels = jnp.take(train_labels, indices)
    bsz = neighbors_labels.shape[0]
    w = jax.nn.softmax(topk_sims / _TEMP, axis=1)
    one_hot = jax.nn.one_hot(neighbors_labels, _NCLS, dtype=topk_sims.dtype)
    m = one_hot * w.reshape(bsz, -1, 1)
    return tuple(jnp.sum(m[:, :k, :], axis=1) for k in _NB_KNN)


# speedup vs baseline: 11.2110x; 11.2110x over previous
import functools

import jax
import jax.numpy as jnp
from jax import lax
from jax.experimental import pallas as pl
from jax.experimental.pallas import tpu as pltpu
from jax.experimental.pallas import tpu_sc as plsc

_TEMP = 0.07
_NCLS = 1000
_Q = 1024
_N = 100000
_D = 1024
_BQ = 1024
_BN = 1024
_NBLK = 98
_NPAD = _NBLK * _BN
_NSC = _NPAD // 128
_NW = 32
_ROWS_PER_W = _Q // _NW
_NCAND = 512
_TCAP = 1024
_BINS = 256
_NEG = -3.0e38
_CLSP = 1024


def _simmat_kernel(x_ref, t_ref, s_ref):
    j = pl.program_id(1)
    s = lax.dot_general(
        x_ref[...], t_ref[...], (((1,), (1,)), ((), ())),
        preferred_element_type=jnp.float32,
        precision=lax.Precision.DEFAULT)
    col = j * _BN + lax.broadcasted_iota(jnp.int32, (_BQ, _BN), 1)
    s_ref[...] = jnp.where(col < _N, s, _NEG).reshape(_BQ, _BN // 128, 128)


def _similarity(features_rank, train_features):
    return pl.pallas_call(
        _simmat_kernel,
        grid=(_Q // _BQ, _NBLK),
        in_specs=[pl.BlockSpec((_BQ, _D), lambda i, j: (i, 0)),
                  pl.BlockSpec((_BN, _D), lambda i, j: (j, 0))],
        out_specs=pl.BlockSpec((_BQ, _BN // 128, 128),
                               lambda i, j: (i, j, 0)),
        out_shape=jax.ShapeDtypeStruct((_Q, _NSC, 128), jnp.float32),
    )(features_rank, train_features)


_CMQ = 32


def _chunkmax128_kernel(s_ref, cm_ref):
    cm_ref[...] = jnp.max(s_ref[...], axis=2)


def _chunkmax128(sims):
    return pl.pallas_call(
        _chunkmax128_kernel,
        grid=(_Q // _CMQ,),
        in_specs=[pl.BlockSpec((_CMQ, _NSC, 128), lambda i: (i, 0, 0))],
        out_specs=pl.BlockSpec((_CMQ, _NSC), lambda i: (i, 0)),
        out_shape=jax.ShapeDtypeStruct((_Q, _NSC), jnp.float32),
    )(sims)


def _sc_vote(sims4, cm, labels4):
    mesh = plsc.VectorSubcoreMesh(core_axis_name="c", subcore_axis_name="s")
    out_t = [jax.ShapeDtypeStruct((_Q, _CLSP), jnp.float32)] * 4
    scratch = [
        pltpu.VMEM((_NSC,), jnp.float32),
        pltpu.VMEM((16 * _BINS,), jnp.int32),
        pltpu.VMEM((4, 128), jnp.int32),
        pltpu.VMEM((4, 128), jnp.int32),
        pltpu.VMEM((128, 128), jnp.float32),
        pltpu.VMEM((128, 128), jnp.int32),
        pltpu.VMEM((_TCAP + 16,), jnp.float32),
        pltpu.VMEM((_TCAP + 16,), jnp.int32),
        pltpu.VMEM((16 * 4 * _CLSP,), jnp.float32),
        pltpu.VMEM((4 * _CLSP,), jnp.float32),
        pltpu.SemaphoreType.DMA,
    ]

    @functools.partial(pl.kernel, mesh=mesh, out_type=out_t,
                       scratch_types=scratch,
                       compiler_params=pltpu.CompilerParams(
                           needs_layout_passes=False))
    def body(sims_ref, cm_ref, lbl_ref, o10, o20, o100, o200,
             cmrow, hist, cidx, lidx, cs, cl, tval, tlbl, outsub, outrow,
             sem):
        wid = lax.axis_index("s") * 2 + lax.axis_index("c")
        lane = lax.broadcasted_iota(jnp.int32, (16,), 0)
        zf = jnp.zeros((16,), jnp.float32)
        zi = jnp.zeros((16,), jnp.int32)
        ones_i = jnp.ones((16,), jnp.int32)

        def z_outsub(i, c):
            outsub[pl.ds(i * 16, 16)] = zf
            return c
        lax.fori_loop(0, (16 * 4 * _CLSP) // 16, z_outsub, 0)

        def z_idx(i, c):
            cidx[i // 8, pl.ds((i % 8) * 16, 16)] = zi
            lidx[i // 8, pl.ds((i % 8) * 16, 16)] = zi
            return c
        lax.fori_loop(0, 32, z_idx, 0)

        def z_hist(i, c):
            hist[pl.ds(i * 16, 16)] = zi
            return c

        def recip16(v):
            r = plsc.bitcast(
                (zi + 0x7EB53567) - plsc.bitcast(v, jnp.int32),
                jnp.float32)
            for _ in range(3):
                r = r * (2.0 - v * r)
            return r

        def merge200(t, carry):
            total, bsel = carry
            g = 15 - t
            s = hist[pl.ds(g * 16, 16)]
            for l in range(1, 16):
                s = s + hist[pl.ds(l * _BINS + g * 16, 16)]
            cr = lax.rev(jnp.cumsum(lax.rev(s, (0,)), axis=0), (0,)) + total
            cand = jnp.where(cr >= 200, lane + g * 16, -1)
            return jnp.max(cr), jnp.maximum(bsel, jnp.max(cand))

        def row_body(r, _unused):
            q = wid * _ROWS_PER_W + r
            pltpu.sync_copy(cm_ref.at[q], cmrow)
            lax.fori_loop(0, (16 * _BINS) // 16, z_hist, 0)

            def rng(i, c):
                mxv, mnv = c
                v = cmrow[pl.ds(i * 16, 16)]
                return (jnp.maximum(mxv, v),
                        jnp.minimum(mnv, jnp.where(v > -2.0, v, 2.0)))
            mxv, mnv = lax.fori_loop(
                0, _NSC // 16, rng,
                (jnp.full((16,), _NEG, jnp.float32),
                 jnp.full((16,), 2.0, jnp.float32)))
            mx = jnp.max(mxv)
            mn = jnp.min(mnv)
            binw1 = jnp.maximum(mx - mn, 1e-20) * (1.0 / _BINS)
            scale1 = recip16(zf + binw1)

            def s1(i, c):
                v = cmrow[pl.ds(i * 16, 16)]
                b = jnp.clip((v - mn) * scale1,
                             0.0, _BINS - 1.0).astype(jnp.int32)
                plsc.addupdate_scatter(hist, [lane * _BINS + b], ones_i)
                return c
            lax.fori_loop(0, _NSC // 16, s1, 0)

            _, b200 = lax.fori_loop(0, 16, merge200,
                                    (jnp.int32(0), jnp.int32(-1)))
            tau = mn + (b200.astype(jnp.float32) - 0.01) * binw1
            tauv = zf + tau

            def c1(i, posv):
                v = cmrow[pl.ds(i * 16, 16)]
                m = v >= tauv
                pvec = posv + jnp.cumsum(m.astype(jnp.int32), axis=0) - 1
                m = jnp.logical_and(m, pvec < _NCAND)
                plsc.store_scatter(cidx, [pvec // 128, pvec % 128],
                                   q * _NSC + i * 16 + lane, mask=m)
                plsc.store_scatter(lidx, [pvec // 128, pvec % 128],
                                   i * 16 + lane, mask=m)
                return posv + plsc.all_reduce_population_count(m)
            posv = lax.fori_loop(0, _NSC // 16, c1, zi)
            pos = jnp.max(posv)
            nw = (pos + 127) // 128

            lax.fori_loop(0, (16 * _BINS) // 16, z_hist, 0)
            binw = jnp.maximum(mx - tau, 1e-20) * (1.0 / _BINS)
            scalev = recip16(zf + binw)

            def wave_a(g, c):
                pltpu.async_copy(sims_ref.at[cidx.at[g]], cs, sem).wait()
                nv = jnp.minimum(pos - g * 128, 128) * 8

                def s2(v, c2):
                    x = cs[v // 8, pl.ds((v % 8) * 16, 16)]
                    b = jnp.clip((x - tau) * scalev,
                                 0.0, _BINS - 1.0).astype(jnp.int32)
                    plsc.addupdate_scatter(hist, [lane * _BINS + b], ones_i)
                    return c2
                return lax.fori_loop(0, nv, s2, c)
            lax.fori_loop(0, nw, wave_a, 0)

            _, b2 = lax.fori_loop(0, 16, merge200,
                                  (jnp.int32(0), jnp.int32(-1)))
            edge_t = tau + (b2.astype(jnp.float32) - 0.01) * binw

            def z_t(i, c):
                tval[pl.ds(i * 16, 16)] = zf - 2.0
                return c
            lax.fori_loop(0, (_TCAP + 16) // 16, z_t, 0)
            ev = zf + edge_t

            def wave_b(g, tpos):
                pltpu.async_copy(sims_ref.at[cidx.at[g]], cs, sem).wait()
                pltpu.async_copy(lbl_ref.at[lidx.at[g]], cl, sem).wait()
                nv = jnp.minimum(pos - g * 128, 128) * 8

                def c2(v, pv):
                    x = cs[v // 8, pl.ds((v % 8) * 16, 16)]
                    lb = cl[v // 8, pl.ds((v % 8) * 16, 16)]
                    m = x >= ev
                    pvec = pv + jnp.cumsum(m.astype(jnp.int32), axis=0) - 1
                    m = jnp.logical_and(m, pvec < _TCAP)
                    plsc.store_scatter(tval, [pvec], x, mask=m)
                    plsc.store_scatter(tlbl, [pvec], lb, mask=m)
                    return pv + plsc.all_reduce_population_count(m)
                return lax.fori_loop(0, nv, c2, tpos)
            tposv = lax.fori_loop(0, nw, wave_b, zi)
            tpos = jnp.max(tposv)
            nt = (tpos + 15) // 16

            def count_ge(t):
                tv = zf + t

                def cb(v, c):
                    x = tval[pl.ds(v * 16, 16)]
                    return c + (x >= tv).astype(jnp.int32)
                return jnp.sum(lax.fori_loop(0, nt, cb, zi))

            def bisect(k):
                def it(_, lh):
                    lo, hi = lh
                    mid = 0.5 * (lo + hi)
                    big = count_ge(mid) >= k
                    return (jnp.where(big, mid, lo),
                            jnp.where(big, hi, mid))
                lo, _hi = lax.fori_loop(0, 30, it, (edge_t, mx + 1e-3))
                return lo
            t10 = bisect(10)
            t20 = bisect(20)
            t100 = bisect(100)
            t200 = bisect(200)
            t10v = zf + t10
            t20v = zf + t20
            t100v = zf + t100
            t200v = zf + t200
            itmp = 1.0 / _TEMP

            def zacc(v, z):
                x = tval[pl.ds(v * 16, 16)]
                return z + jnp.where(x >= t200v,
                                     jnp.exp((x - mx) * itmp), 0.0)
            zvec = lax.fori_loop(0, nt, zacc, zf)
            zinvv = recip16(zf + jnp.sum(zvec))

            def band_addr(x, lb):
                bidx = 3 - ((x >= t10v).astype(jnp.int32)
                            + (x >= t20v).astype(jnp.int32)
                            + (x >= t100v).astype(jnp.int32))
                return lane * (4 * _CLSP) + bidx * _CLSP + lb

            def vote(v, c):
                x = tval[pl.ds(v * 16, 16)]
                lb = tlbl[pl.ds(v * 16, 16)]
                sel = x >= t200v
                w = jnp.exp((x - mx) * itmp) * zinvv
                plsc.addupdate_scatter(outsub, [band_addr(x, lb)], w,
                                       mask=sel)
                return c
            lax.fori_loop(0, nt, vote, 0)

            def mo(u, c):
                acc = zf
                for bb in range(4):
                    s = outsub[pl.ds(bb * _CLSP + u * 16, 16)]
                    for l in range(1, 16):
                        s = s + outsub[pl.ds(l * (4 * _CLSP)
                                             + bb * _CLSP + u * 16, 16)]
                    acc = acc + s
                    outrow[pl.ds(bb * _CLSP + u * 16, 16)] = acc
                return c
            lax.fori_loop(0, _CLSP // 16, mo, 0)

            def unvote(v, c):
                x = tval[pl.ds(v * 16, 16)]
                lb = tlbl[pl.ds(v * 16, 16)]
                sel = x >= t200v
                plsc.store_scatter(outsub, [band_addr(x, lb)], zf,
                                   mask=sel)
                return c
            lax.fori_loop(0, nt, unvote, 0)

            pltpu.sync_copy(outrow.at[pl.ds(0 * _CLSP, _CLSP)], o10.at[q])
            pltpu.sync_copy(outrow.at[pl.ds(1 * _CLSP, _CLSP)], o20.at[q])
            pltpu.sync_copy(outrow.at[pl.ds(2 * _CLSP, _CLSP)], o100.at[q])
            pltpu.sync_copy(outrow.at[pl.ds(3 * _CLSP, _CLSP)], o200.at[q])
            return _unused

        lax.fori_loop(0, _ROWS_PER_W, row_body, 0)

    return body(sims4, cm, labels4)


def kernel(features_rank, train_features, train_labels):
    sims = _similarity(features_rank, train_features)
    cm = _chunkmax128(sims)
    sims4 = sims.reshape(_Q * _NSC, 128)
    labels4 = jnp.pad(train_labels, (0, _NPAD - _N)).reshape(_NSC, 128)
    o10, o20, o100, o200 = _sc_vote(sims4, cm, labels4)
    return (o10[:, :_NCLS], o20[:, :_NCLS], o100[:, :_NCLS], o200[:, :_NCLS])


# speedup vs baseline: 11.3016x; 1.0081x over previous
import functools

import jax
import jax.numpy as jnp
from jax import lax
from jax.experimental import pallas as pl
from jax.experimental.pallas import tpu as pltpu
from jax.experimental.pallas import tpu_sc as plsc

_TEMP = 0.07
_NCLS = 1000
_Q = 1024
_N = 100000
_D = 1024
_BQ = 1024
_BN = 1024
_NBLK = 98
_NPAD = _NBLK * _BN
_NSC = _NPAD // 128
_NW = 32
_ROWS_PER_W = _Q // _NW
_NCAND = 512
_TCAP = 1024
_BINS = 256
_HP = 257
_NEG = -3.0e38
_CLSP = 1024
_LP = 4 * _CLSP + 1


def _simmat_kernel(x_ref, t_ref, s_ref):
    j = pl.program_id(1)
    s = lax.dot_general(
        x_ref[...], t_ref[...], (((1,), (1,)), ((), ())),
        preferred_element_type=jnp.float32,
        precision=lax.Precision.DEFAULT)
    col = j * _BN + lax.broadcasted_iota(jnp.int32, (_BQ, _BN), 1)
    s_ref[...] = jnp.where(col < _N, s, _NEG).reshape(_BQ, _BN // 128, 128)


def _similarity(features_rank, train_features):
    return pl.pallas_call(
        _simmat_kernel,
        grid=(_Q // _BQ, _NBLK),
        in_specs=[pl.BlockSpec((_BQ, _D), lambda i, j: (i, 0)),
                  pl.BlockSpec((_BN, _D), lambda i, j: (j, 0))],
        out_specs=pl.BlockSpec((_BQ, _BN // 128, 128),
                               lambda i, j: (i, j, 0)),
        out_shape=jax.ShapeDtypeStruct((_Q, _NSC, 128), jnp.float32),
    )(features_rank, train_features)


_CMQ = 32


def _chunkmax128_kernel(s_ref, cm_ref):
    cm_ref[...] = jnp.max(s_ref[...], axis=2)


def _chunkmax128(sims):
    return pl.pallas_call(
        _chunkmax128_kernel,
        grid=(_Q // _CMQ,),
        in_specs=[pl.BlockSpec((_CMQ, _NSC, 128), lambda i: (i, 0, 0))],
        out_specs=pl.BlockSpec((_CMQ, _NSC), lambda i: (i, 0)),
        out_shape=jax.ShapeDtypeStruct((_Q, _NSC), jnp.float32),
    )(sims)


def _sc_vote(sims4, cm, labels4):
    mesh = plsc.VectorSubcoreMesh(core_axis_name="c", subcore_axis_name="s")
    out_t = [jax.ShapeDtypeStruct((_Q, _CLSP), jnp.float32)] * 4
    scratch = [
        pltpu.VMEM((_NSC,), jnp.float32),
        pltpu.VMEM((16 * _HP + 16,), jnp.int32),
        pltpu.VMEM((4, 128), jnp.int32),
        pltpu.VMEM((4, 128), jnp.int32),
        pltpu.VMEM((128, 128), jnp.float32),
        pltpu.VMEM((128, 128), jnp.int32),
        pltpu.VMEM((_TCAP + 16,), jnp.float32),
        pltpu.VMEM((_TCAP + 16,), jnp.int32),
        pltpu.VMEM((16 * _LP + 16,), jnp.float32),
        pltpu.VMEM((4 * _CLSP,), jnp.float32),
        pltpu.SemaphoreType.DMA,
    ]

    @functools.partial(pl.kernel, mesh=mesh, out_type=out_t,
                       scratch_types=scratch,
                       compiler_params=pltpu.CompilerParams(
                           needs_layout_passes=False))
    def body(sims_ref, cm_ref, lbl_ref, o10, o20, o100, o200,
             cmrow, hist, cidx, lidx, cs, cl, tval, tlbl, outsub, outrow,
             sem):
        wid = lax.axis_index("s") * 2 + lax.axis_index("c")
        lane = lax.broadcasted_iota(jnp.int32, (16,), 0)
        zf = jnp.zeros((16,), jnp.float32)
        zi = jnp.zeros((16,), jnp.int32)
        ones_i = jnp.ones((16,), jnp.int32)

        def z_outsub(i, c):
            outsub[pl.ds(i * 16, 16)] = zf
            return c
        lax.fori_loop(0, (16 * _LP + 16) // 16, z_outsub, 0)

        def z_idx(i, c):
            cidx[i // 8, pl.ds((i % 8) * 16, 16)] = zi
            lidx[i // 8, pl.ds((i % 8) * 16, 16)] = zi
            return c
        lax.fori_loop(0, 32, z_idx, 0)

        def z_hist(i, c):
            hist[pl.ds(i * 16, 16)] = zi
            return c

        def recip16(v):
            r = plsc.bitcast(
                (zi + 0x7EB53567) - plsc.bitcast(v, jnp.int32),
                jnp.float32)
            for _ in range(3):
                r = r * (2.0 - v * r)
            return r

        def merge200(t, carry):
            total, bsel = carry
            g = 15 - t
            s = hist[pl.ds(g * 16, 16)]
            for l in range(1, 16):
                s = s + hist[pl.ds(l * _HP + g * 16, 16)]
            cr = lax.rev(jnp.cumsum(lax.rev(s, (0,)), axis=0), (0,)) + total
            cand = jnp.where(cr >= 200, lane + g * 16, -1)
            return jnp.max(cr), jnp.maximum(bsel, jnp.max(cand))

        def row_body(r, _unused):
            q = wid * _ROWS_PER_W + r
            pltpu.sync_copy(cm_ref.at[q], cmrow)
            lax.fori_loop(0, (16 * _HP + 15) // 16, z_hist, 0)

            def rng(i, c):
                mxv, mnv = c
                v = cmrow[pl.ds(i * 16, 16)]
                return (jnp.maximum(mxv, v),
                        jnp.minimum(mnv, jnp.where(v > -2.0, v, 2.0)))
            mxv, mnv = lax.fori_loop(
                0, _NSC // 16, rng,
                (jnp.full((16,), _NEG, jnp.float32),
                 jnp.full((16,), 2.0, jnp.float32)))
            mx = jnp.max(mxv)
            mn = jnp.min(mnv)
            binw1 = jnp.maximum(mx - mn, 1e-20) * (1.0 / _BINS)
            scale1 = recip16(zf + binw1)

            def s1(i, c):
                v = cmrow[pl.ds(i * 16, 16)]
                b = jnp.clip((v - mn) * scale1,
                             0.0, _BINS - 1.0).astype(jnp.int32)
                plsc.addupdate_scatter(hist, [lane * _HP + b], ones_i)
                return c
            lax.fori_loop(0, _NSC // 16, s1, 0)

            _, b200 = lax.fori_loop(0, 16, merge200,
                                    (jnp.int32(0), jnp.int32(-1)))
            tau = mn + (b200.astype(jnp.float32) - 0.01) * binw1
            tauv = zf + tau

            def c1(i, posv):
                v = cmrow[pl.ds(i * 16, 16)]
                m = v >= tauv
                pvec = posv + jnp.cumsum(m.astype(jnp.int32), axis=0) - 1
                m = jnp.logical_and(m, pvec < _NCAND)
                plsc.store_scatter(cidx, [pvec // 128, pvec % 128],
                                   q * _NSC + i * 16 + lane, mask=m)
                plsc.store_scatter(lidx, [pvec // 128, pvec % 128],
                                   i * 16 + lane, mask=m)
                return posv + plsc.all_reduce_population_count(m)
            posv = lax.fori_loop(0, _NSC // 16, c1, zi)
            pos = jnp.max(posv)
            nw = (pos + 127) // 128

            lax.fori_loop(0, (16 * _HP + 15) // 16, z_hist, 0)
            binw = jnp.maximum(mx - tau, 1e-20) * (1.0 / _BINS)
            scalev = recip16(zf + binw)

            def wave_a(g, c):
                pltpu.async_copy(sims_ref.at[cidx.at[g]], cs, sem).wait()
                nv = jnp.minimum(pos - g * 128, 128)

                def s2(v, c2):
                    for e in range(8):
                        x = cs[v, pl.ds(e * 16, 16)]
                        b = jnp.clip((x - tau) * scalev,
                                     0.0, _BINS - 1.0).astype(jnp.int32)
                        plsc.addupdate_scatter(hist, [lane * _HP + b],
                                               ones_i)
                    return c2
                return lax.fori_loop(0, nv, s2, c)
            lax.fori_loop(0, nw, wave_a, 0)

            _, b2 = lax.fori_loop(0, 16, merge200,
                                  (jnp.int32(0), jnp.int32(-1)))
            edge_t = tau + (b2.astype(jnp.float32) - 0.01) * binw

            def z_t(i, c):
                tval[pl.ds(i * 16, 16)] = zf - 2.0
                return c
            lax.fori_loop(0, (_TCAP + 16) // 16, z_t, 0)
            ev = zf + edge_t

            def wave_b(g, tpos):
                pltpu.async_copy(sims_ref.at[cidx.at[g]], cs, sem).wait()
                pltpu.async_copy(lbl_ref.at[lidx.at[g]], cl, sem).wait()
                nv = jnp.minimum(pos - g * 128, 128)

                def c2(v, pv):
                    for e in range(8):
                        x = cs[v, pl.ds(e * 16, 16)]
                        lb = cl[v, pl.ds(e * 16, 16)]
                        m = x >= ev
                        pvec = pv + jnp.cumsum(m.astype(jnp.int32),
                                               axis=0) - 1
                        m = jnp.logical_and(m, pvec < _TCAP)
                        plsc.store_scatter(tval, [pvec], x, mask=m)
                        plsc.store_scatter(tlbl, [pvec], lb, mask=m)
                        pv = pv + plsc.all_reduce_population_count(m)
                    return pv
                return lax.fori_loop(0, nv, c2, tpos)
            tposv = lax.fori_loop(0, nw, wave_b, zi)
            tpos = jnp.max(tposv)
            nt = (tpos + 15) // 16

            def count_ge(t):
                tv = zf + t

                def cb(v, c):
                    x = tval[pl.ds(v * 16, 16)]
                    return c + (x >= tv).astype(jnp.int32)
                return jnp.sum(lax.fori_loop(0, nt, cb, zi))

            def bisect(k):
                def it(_, lh):
                    lo, hi = lh
                    mid = 0.5 * (lo + hi)
                    big = count_ge(mid) >= k
                    return (jnp.where(big, mid, lo),
                            jnp.where(big, hi, mid))
                lo, _hi = lax.fori_loop(0, 30, it, (edge_t, mx + 1e-3))
                return lo
            t10 = bisect(10)
            t20 = bisect(20)
            t100 = bisect(100)
            t200 = bisect(200)
            t10v = zf + t10
            t20v = zf + t20
            t100v = zf + t100
            t200v = zf + t200
            itmp = 1.0 / _TEMP

            def zacc(v, z):
                x = tval[pl.ds(v * 16, 16)]
                return z + jnp.where(x >= t200v,
                                     jnp.exp((x - mx) * itmp), 0.0)
            zvec = lax.fori_loop(0, nt, zacc, zf)
            zinvv = recip16(zf + jnp.sum(zvec))

            def band_addr(x, lb):
                bidx = 3 - ((x >= t10v).astype(jnp.int32)
                            + (x >= t20v).astype(jnp.int32)
                            + (x >= t100v).astype(jnp.int32))
                return lane * _LP + bidx * _CLSP + lb

            def vote(v, c):
                x = tval[pl.ds(v * 16, 16)]
                lb = tlbl[pl.ds(v * 16, 16)]
                sel = x >= t200v
                w = jnp.exp((x - mx) * itmp) * zinvv
                plsc.addupdate_scatter(outsub, [band_addr(x, lb)], w,
                                       mask=sel)
                return c
            lax.fori_loop(0, nt, vote, 0)

            def mo(u, c):
                acc = zf
                for bb in range(4):
                    s = outsub[pl.ds(bb * _CLSP + u * 16, 16)]
                    for l in range(1, 16):
                        s = s + outsub[pl.ds(l * _LP
                                             + bb * _CLSP + u * 16, 16)]
                    acc = acc + s
                    outrow[pl.ds(bb * _CLSP + u * 16, 16)] = acc
                return c
            lax.fori_loop(0, _CLSP // 16, mo, 0)

            def unvote(v, c):
                x = tval[pl.ds(v * 16, 16)]
                lb = tlbl[pl.ds(v * 16, 16)]
                sel = x >= t200v
                plsc.store_scatter(outsub, [band_addr(x, lb)], zf,
                                   mask=sel)
                return c
            lax.fori_loop(0, nt, unvote, 0)

            pltpu.sync_copy(outrow.at[pl.ds(0 * _CLSP, _CLSP)], o10.at[q])
            pltpu.sync_copy(outrow.at[pl.ds(1 * _CLSP, _CLSP)], o20.at[q])
            pltpu.sync_copy(outrow.at[pl.ds(2 * _CLSP, _CLSP)], o100.at[q])
            pltpu.sync_copy(outrow.at[pl.ds(3 * _CLSP, _CLSP)], o200.at[q])
            return _unused

        lax.fori_loop(0, _ROWS_PER_W, row_body, 0)

    return body(sims4, cm, labels4)


def kernel(features_rank, train_features, train_labels):
    sims = _similarity(features_rank, train_features)
    cm = _chunkmax128(sims)
    sims4 = sims.reshape(_Q * _NSC, 128)
    labels4 = jnp.pad(train_labels, (0, _NPAD - _N)).reshape(_NSC, 128)
    o10, o20, o100, o200 = _sc_vote(sims4, cm, labels4)
    return (o10[:, :_NCLS], o20[:, :_NCLS], o100[:, :_NCLS], o200[:, :_NCLS])


# speedup vs baseline: 19.2983x; 1.7076x over previous
import functools

import jax
import jax.numpy as jnp
from jax import lax
from jax.experimental import pallas as pl
from jax.experimental.pallas import tpu as pltpu
from jax.experimental.pallas import tpu_sc as plsc

_TEMP = 0.07
_NCLS = 1000
_Q = 1024
_N = 100000
_D = 1024
_BQ = 1024
_BN = 1024
_NBLK = 98
_NPAD = _NBLK * _BN
_NSC = _NPAD // 128
_NW = 32
_ROWS_PER_W = _Q // _NW
_NCAND = 512
_TCAP = 1024
_BINS = 256
_HP = 257
_NEG = -3.0e38
_CLSP = 1024
_LP = 4 * _CLSP + 1


def _simmat_kernel(x_ref, t_ref, s_ref):
    j = pl.program_id(1)
    s = lax.dot_general(
        x_ref[...], t_ref[...], (((1,), (1,)), ((), ())),
        preferred_element_type=jnp.float32,
        precision=lax.Precision.DEFAULT)
    col = j * _BN + lax.broadcasted_iota(jnp.int32, (_BQ, _BN), 1)
    s_ref[...] = jnp.where(col < _N, s, _NEG).reshape(_BQ, _BN // 128, 128)


def _similarity(features_rank, train_features):
    return pl.pallas_call(
        _simmat_kernel,
        grid=(_Q // _BQ, _NBLK),
        in_specs=[pl.BlockSpec((_BQ, _D), lambda i, j: (i, 0)),
                  pl.BlockSpec((_BN, _D), lambda i, j: (j, 0))],
        out_specs=pl.BlockSpec((_BQ, _BN // 128, 128),
                               lambda i, j: (i, j, 0)),
        out_shape=jax.ShapeDtypeStruct((_Q, _NSC, 128), jnp.float32),
    )(features_rank, train_features)


_CMQ = 32


def _chunkmax128_kernel(s_ref, cm_ref):
    cm_ref[...] = jnp.max(s_ref[...], axis=2)


def _chunkmax128(sims):
    return pl.pallas_call(
        _chunkmax128_kernel,
        grid=(_Q // _CMQ,),
        in_specs=[pl.BlockSpec((_CMQ, _NSC, 128), lambda i: (i, 0, 0))],
        out_specs=pl.BlockSpec((_CMQ, _NSC), lambda i: (i, 0)),
        out_shape=jax.ShapeDtypeStruct((_Q, _NSC), jnp.float32),
    )(sims)


def _sc_vote(sims4, cm, labels4):
    mesh = plsc.VectorSubcoreMesh(core_axis_name="c", subcore_axis_name="s")
    out_t = [jax.ShapeDtypeStruct((_Q, _CLSP), jnp.float32)] * 4
    scratch = [
        pltpu.VMEM((_NSC,), jnp.float32),
        pltpu.VMEM((16 * _HP + 16,), jnp.int32),
        pltpu.VMEM((4, 128), jnp.int32),
        pltpu.VMEM((4, 128), jnp.int32),
        pltpu.VMEM((128, 128), jnp.float32),
        pltpu.VMEM((128, 128), jnp.int32),
        pltpu.VMEM((_TCAP + 16,), jnp.float32),
        pltpu.VMEM((_TCAP + 16,), jnp.int32),
        pltpu.VMEM((16 * _LP + 16,), jnp.float32),
        pltpu.VMEM((4 * _CLSP,), jnp.float32),
        pltpu.SemaphoreType.DMA,
    ]

    @functools.partial(pl.kernel, mesh=mesh, out_type=out_t,
                       scratch_types=scratch,
                       compiler_params=pltpu.CompilerParams(
                           needs_layout_passes=False))
    def body(sims_ref, cm_ref, lbl_ref, o10, o20, o100, o200,
             cmrow, hist, cidx, lidx, cs, cl, tval, tlbl, outsub, outrow,
             sem):
        wid = lax.axis_index("s") * 2 + lax.axis_index("c")
        lane = lax.broadcasted_iota(jnp.int32, (16,), 0)
        zf = jnp.zeros((16,), jnp.float32)
        zi = jnp.zeros((16,), jnp.int32)
        ones_i = jnp.ones((16,), jnp.int32)

        def z_outsub(i, c):
            outsub[pl.ds(i * 16, 16)] = zf
            return c
        lax.fori_loop(0, (16 * _LP + 16) // 16, z_outsub, 0)

        def z_idx(i, c):
            cidx[i // 8, pl.ds((i % 8) * 16, 16)] = zi
            lidx[i // 8, pl.ds((i % 8) * 16, 16)] = zi
            return c
        lax.fori_loop(0, 32, z_idx, 0)

        def z_hist(i, c):
            hist[pl.ds(i * 16, 16)] = zi
            return c

        def recip16(v):
            r = plsc.bitcast(
                (zi + 0x7EB53567) - plsc.bitcast(v, jnp.int32),
                jnp.float32)
            for _ in range(3):
                r = r * (2.0 - v * r)
            return r

        def merge200(t, carry):
            total, bsel = carry
            g = 15 - t
            s = hist[pl.ds(g * 16, 16)]
            for l in range(1, 16):
                s = s + hist[pl.ds(l * _HP + g * 16, 16)]
            cr = lax.rev(jnp.cumsum(lax.rev(s, (0,)), axis=0), (0,)) + total
            cand = jnp.where(cr >= 200, lane + g * 16, -1)
            return jnp.max(cr), jnp.maximum(bsel, jnp.max(cand))

        def row_body(r, _unused):
            q = wid * _ROWS_PER_W + r
            pltpu.sync_copy(cm_ref.at[q], cmrow)
            lax.fori_loop(0, (16 * _HP + 15) // 16, z_hist, 0)

            def rng(i, c):
                mxv, mnv = c
                v = cmrow[pl.ds(i * 16, 16)]
                return (jnp.maximum(mxv, v),
                        jnp.minimum(mnv, jnp.where(v > -2.0, v, 2.0)))
            mxv, mnv = lax.fori_loop(
                0, _NSC // 16, rng,
                (jnp.full((16,), _NEG, jnp.float32),
                 jnp.full((16,), 2.0, jnp.float32)))
            mx = jnp.max(mxv)
            mn = jnp.min(mnv)
            binw1 = jnp.maximum(mx - mn, 1e-20) * (1.0 / _BINS)
            scale1 = recip16(zf + binw1)

            def s1(i, c):
                v = cmrow[pl.ds(i * 16, 16)]
                b = jnp.clip((v - mn) * scale1,
                             0.0, _BINS - 1.0).astype(jnp.int32)
                plsc.addupdate_scatter(hist, [lane * _HP + b], ones_i)
                return c
            lax.fori_loop(0, _NSC // 16, s1, 0)

            _, b200 = lax.fori_loop(0, 16, merge200,
                                    (jnp.int32(0), jnp.int32(-1)))
            tau = mn + (b200.astype(jnp.float32) - 0.01) * binw1
            tauv = zf + tau

            def c1(i, posv):
                v = cmrow[pl.ds(i * 16, 16)]
                m = v >= tauv
                pvec = posv + jnp.cumsum(m.astype(jnp.int32), axis=0) - 1
                m = jnp.logical_and(m, pvec < _NCAND)
                plsc.store_scatter(cidx, [pvec // 128, pvec % 128],
                                   q * _NSC + i * 16 + lane, mask=m)
                plsc.store_scatter(lidx, [pvec // 128, pvec % 128],
                                   i * 16 + lane, mask=m)
                return posv + plsc.all_reduce_population_count(m)
            posv = lax.fori_loop(0, _NSC // 16, c1, zi)
            pos = jnp.max(posv)
            nw = (pos + 127) // 128

            def z_t(i, c):
                tval[pl.ds(i * 16, 16)] = zf - 2.0
                return c
            lax.fori_loop(0, (_TCAP + 16) // 16, z_t, 0)
            edge_t = tau
            ev = tauv

            def wave_b(g, tpos):
                pltpu.async_copy(sims_ref.at[cidx.at[g]], cs, sem).wait()
                pltpu.async_copy(lbl_ref.at[lidx.at[g]], cl, sem).wait()
                nv = jnp.minimum(pos - g * 128, 128)

                def c2(v, pv):
                    for e in range(8):
                        x = cs[v, pl.ds(e * 16, 16)]
                        lb = cl[v, pl.ds(e * 16, 16)]
                        m = x >= ev
                        pvec = pv + jnp.cumsum(m.astype(jnp.int32),
                                               axis=0) - 1
                        m = jnp.logical_and(m, pvec < _TCAP)
                        plsc.store_scatter(tval, [pvec], x, mask=m)
                        plsc.store_scatter(tlbl, [pvec], lb, mask=m)
                        pv = pv + plsc.all_reduce_population_count(m)
                    return pv
                return lax.fori_loop(0, nv, c2, tpos)
            tposv = lax.fori_loop(0, nw, wave_b, zi)
            tpos = jnp.max(tposv)
            nt = (tpos + 15) // 16

            def count_ge(t):
                tv = zf + t

                def cb(v, c):
                    x = tval[pl.ds(v * 16, 16)]
                    return c + (x >= tv).astype(jnp.int32)
                return jnp.sum(lax.fori_loop(0, nt, cb, zi))

            def bisect(k):
                def it(_, lh):
                    lo, hi = lh
                    mid = 0.5 * (lo + hi)
                    big = count_ge(mid) >= k
                    return (jnp.where(big, mid, lo),
                            jnp.where(big, hi, mid))
                lo, _hi = lax.fori_loop(0, 30, it, (edge_t, mx + 1e-3))
                return lo
            t10 = bisect(10)
            t20 = bisect(20)
            t100 = bisect(100)
            t200 = bisect(200)
            t10v = zf + t10
            t20v = zf + t20
            t100v = zf + t100
            t200v = zf + t200
            itmp = 1.0 / _TEMP

            def zacc(v, z):
                x = tval[pl.ds(v * 16, 16)]
                return z + jnp.where(x >= t200v,
                                     jnp.exp((x - mx) * itmp), 0.0)
            zvec = lax.fori_loop(0, nt, zacc, zf)
            zinvv = recip16(zf + jnp.sum(zvec))

            def band_addr(x, lb):
                bidx = 3 - ((x >= t10v).astype(jnp.int32)
                            + (x >= t20v).astype(jnp.int32)
                            + (x >= t100v).astype(jnp.int32))
                return lane * _LP + bidx * _CLSP + lb

            def vote(v, c):
                x = tval[pl.ds(v * 16, 16)]
                lb = tlbl[pl.ds(v * 16, 16)]
                sel = x >= t200v
                w = jnp.exp((x - mx) * itmp) * zinvv
                plsc.addupdate_scatter(outsub, [band_addr(x, lb)], w,
                                       mask=sel)
                return c
            lax.fori_loop(0, nt, vote, 0)

            def mo(u, c):
                acc = zf
                for bb in range(4):
                    s = outsub[pl.ds(bb * _CLSP + u * 16, 16)]
                    for l in range(1, 16):
                        s = s + outsub[pl.ds(l * _LP
                                             + bb * _CLSP + u * 16, 16)]
                    acc = acc + s
                    outrow[pl.ds(bb * _CLSP + u * 16, 16)] = acc
                return c
            lax.fori_loop(0, _CLSP // 16, mo, 0)

            def unvote(v, c):
                x = tval[pl.ds(v * 16, 16)]
                lb = tlbl[pl.ds(v * 16, 16)]
                sel = x >= t200v
                plsc.store_scatter(outsub, [band_addr(x, lb)], zf,
                                   mask=sel)
                return c
            lax.fori_loop(0, nt, unvote, 0)

            pltpu.sync_copy(outrow.at[pl.ds(0 * _CLSP, _CLSP)], o10.at[q])
            pltpu.sync_copy(outrow.at[pl.ds(1 * _CLSP, _CLSP)], o20.at[q])
            pltpu.sync_copy(outrow.at[pl.ds(2 * _CLSP, _CLSP)], o100.at[q])
            pltpu.sync_copy(outrow.at[pl.ds(3 * _CLSP, _CLSP)], o200.at[q])
            return _unused

        lax.fori_loop(0, _ROWS_PER_W, row_body, 0)

    return body(sims4, cm, labels4)


def kernel(features_rank, train_features, train_labels):
    sims = _similarity(features_rank, train_features)
    cm = _chunkmax128(sims)
    sims4 = sims.reshape(_Q * _NSC, 128)
    labels4 = jnp.pad(train_labels, (0, _NPAD - _N)).reshape(_NSC, 128)
    o10, o20, o100, o200 = _sc_vote(sims4, cm, labels4)
    return (o10[:, :_NCLS], o20[:, :_NCLS], o100[:, :_NCLS], o200[:, :_NCLS])


# speedup vs baseline: 19.7768x; 1.0248x over previous
import functools

import jax
import jax.numpy as jnp
from jax import lax
from jax.experimental import pallas as pl
from jax.experimental.pallas import tpu as pltpu
from jax.experimental.pallas import tpu_sc as plsc

_TEMP = 0.07
_NCLS = 1000
_Q = 1024
_N = 100000
_D = 1024
_BQ = 1024
_BN = 1024
_NBLK = 98
_NPAD = _NBLK * _BN
_NSC = _NPAD // 128
_NW = 32
_ROWS_PER_W = _Q // _NW
_NCAND = 512
_TCAP = 1024
_BINS = 256
_HP = 257
_NEG = -3.0e38
_CLSP = 1024
_LP = 4 * _CLSP + 1


def _simmat_kernel(x_ref, t_ref, s_ref):
    j = pl.program_id(1)
    s = lax.dot_general(
        x_ref[...], t_ref[...], (((1,), (1,)), ((), ())),
        preferred_element_type=jnp.float32,
        precision=lax.Precision.DEFAULT)
    col = j * _BN + lax.broadcasted_iota(jnp.int32, (_BQ, _BN), 1)
    s_ref[...] = jnp.where(col < _N, s, _NEG).reshape(_BQ, _BN // 128, 128)


def _similarity(features_rank, train_features):
    return pl.pallas_call(
        _simmat_kernel,
        grid=(_Q // _BQ, _NBLK),
        in_specs=[pl.BlockSpec((_BQ, _D), lambda i, j: (i, 0)),
                  pl.BlockSpec((_BN, _D), lambda i, j: (j, 0))],
        out_specs=pl.BlockSpec((_BQ, _BN // 128, 128),
                               lambda i, j: (i, j, 0)),
        out_shape=jax.ShapeDtypeStruct((_Q, _NSC, 128), jnp.float32),
    )(features_rank, train_features)


_CMQ = 32


def _chunkmax128_kernel(s_ref, cm_ref):
    cm_ref[...] = jnp.max(s_ref[...], axis=2)


def _chunkmax128(sims):
    return pl.pallas_call(
        _chunkmax128_kernel,
        grid=(_Q // _CMQ,),
        in_specs=[pl.BlockSpec((_CMQ, _NSC, 128), lambda i: (i, 0, 0))],
        out_specs=pl.BlockSpec((_CMQ, _NSC), lambda i: (i, 0)),
        out_shape=jax.ShapeDtypeStruct((_Q, _NSC), jnp.float32),
    )(sims)


def _sc_vote(sims4, cm, labels4):
    mesh = plsc.VectorSubcoreMesh(core_axis_name="c", subcore_axis_name="s")
    out_t = [jax.ShapeDtypeStruct((_Q, _CLSP), jnp.float32)] * 4
    scratch = [
        pltpu.VMEM((_NSC,), jnp.float32),
        pltpu.VMEM((16 * _HP + 16,), jnp.int32),
        pltpu.VMEM((4, 128), jnp.int32),
        pltpu.VMEM((4, 128), jnp.int32),
        pltpu.VMEM((128, 128), jnp.float32),
        pltpu.VMEM((128, 128), jnp.int32),
        pltpu.VMEM((_TCAP + 16,), jnp.float32),
        pltpu.VMEM((_TCAP + 16,), jnp.int32),
        pltpu.VMEM((16 * _LP + 16,), jnp.float32),
        pltpu.VMEM((4 * _CLSP,), jnp.float32),
        pltpu.SemaphoreType.DMA,
    ]

    @functools.partial(pl.kernel, mesh=mesh, out_type=out_t,
                       scratch_types=scratch,
                       compiler_params=pltpu.CompilerParams(
                           needs_layout_passes=False))
    def body(sims_ref, cm_ref, lbl_ref, o10, o20, o100, o200,
             cmrow, hist, cidx, lidx, cs, cl, tval, tlbl, outsub, outrow,
             sem):
        wid = lax.axis_index("s") * 2 + lax.axis_index("c")
        lane = lax.broadcasted_iota(jnp.int32, (16,), 0)
        zf = jnp.zeros((16,), jnp.float32)
        zi = jnp.zeros((16,), jnp.int32)
        ones_i = jnp.ones((16,), jnp.int32)

        def z_outsub(i, c):
            outsub[pl.ds(i * 16, 16)] = zf
            return c
        lax.fori_loop(0, (16 * _LP + 16) // 16, z_outsub, 0)

        def z_idx(i, c):
            cidx[i // 8, pl.ds((i % 8) * 16, 16)] = zi
            lidx[i // 8, pl.ds((i % 8) * 16, 16)] = zi
            return c
        lax.fori_loop(0, 32, z_idx, 0)

        def z_hist(i, c):
            hist[pl.ds(i * 16, 16)] = zi
            return c

        def recip16(v):
            r = plsc.bitcast(
                (zi + 0x7EB53567) - plsc.bitcast(v, jnp.int32),
                jnp.float32)
            for _ in range(3):
                r = r * (2.0 - v * r)
            return r

        def merge200(t, carry):
            total, bsel = carry
            g = 15 - t
            s = hist[pl.ds(g * 16, 16)]
            for l in range(1, 16):
                s = s + hist[pl.ds(l * _HP + g * 16, 16)]
            cr = lax.rev(jnp.cumsum(lax.rev(s, (0,)), axis=0), (0,)) + total
            cand = jnp.where(cr >= 200, lane + g * 16, -1)
            return jnp.max(cr), jnp.maximum(bsel, jnp.max(cand))

        def row_body(r, _unused):
            q = wid * _ROWS_PER_W + r
            pltpu.sync_copy(cm_ref.at[q], cmrow)
            lax.fori_loop(0, (16 * _HP + 15) // 16, z_hist, 0)

            def rng(i, c):
                mxv, mnv = c
                v = cmrow[pl.ds(i * 16, 16)]
                return (jnp.maximum(mxv, v),
                        jnp.minimum(mnv, jnp.where(v > -2.0, v, 2.0)))
            mxv, mnv = lax.fori_loop(
                0, _NSC // 16, rng,
                (jnp.full((16,), _NEG, jnp.float32),
                 jnp.full((16,), 2.0, jnp.float32)))
            mx = jnp.max(mxv)
            mn = jnp.min(mnv)
            binw1 = jnp.maximum(mx - mn, 1e-20) * (1.0 / _BINS)
            scale1 = recip16(zf + binw1)

            def s1(i, c):
                v = cmrow[pl.ds(i * 16, 16)]
                b = jnp.clip((v - mn) * scale1,
                             0.0, _BINS - 1.0).astype(jnp.int32)
                plsc.addupdate_scatter(hist, [lane * _HP + b], ones_i)
                return c
            lax.fori_loop(0, _NSC // 16, s1, 0)

            _, b200 = lax.fori_loop(0, 16, merge200,
                                    (jnp.int32(0), jnp.int32(-1)))
            tau = mn + (b200.astype(jnp.float32) - 0.01) * binw1
            tauv = zf + tau

            def c1(i, posv):
                v = cmrow[pl.ds(i * 16, 16)]
                m = v >= tauv
                pvec = posv + jnp.cumsum(m.astype(jnp.int32), axis=0) - 1
                m = jnp.logical_and(m, pvec < _NCAND)
                plsc.store_scatter(cidx, [pvec // 128, pvec % 128],
                                   q * _NSC + i * 16 + lane, mask=m)
                plsc.store_scatter(lidx, [pvec // 128, pvec % 128],
                                   i * 16 + lane, mask=m)
                return posv + plsc.all_reduce_population_count(m)
            posv = lax.fori_loop(0, _NSC // 16, c1, zi)
            pos = jnp.max(posv)
            nw = (pos + 127) // 128

            def z_t(i, c):
                tval[pl.ds(i * 16, 16)] = zf - 2.0
                return c
            lax.fori_loop(0, (_TCAP + 16) // 16, z_t, 0)
            edge_t = tau
            ev = tauv

            def wave_b(g, tpos):
                cpy_s = pltpu.async_copy(sims_ref.at[cidx.at[g]], cs, sem)
                cpy_l = pltpu.async_copy(lbl_ref.at[lidx.at[g]], cl, sem)
                cpy_s.wait()
                cpy_l.wait()
                nv = jnp.minimum(pos - g * 128, 128)

                def c2(v, pv):
                    for e in range(8):
                        x = cs[v, pl.ds(e * 16, 16)]
                        lb = cl[v, pl.ds(e * 16, 16)]
                        m = x >= ev
                        pvec = pv + jnp.cumsum(m.astype(jnp.int32),
                                               axis=0) - 1
                        m = jnp.logical_and(m, pvec < _TCAP)
                        plsc.store_scatter(tval, [pvec], x, mask=m)
                        plsc.store_scatter(tlbl, [pvec], lb, mask=m)
                        pv = pv + plsc.all_reduce_population_count(m)
                    return pv
                return lax.fori_loop(0, nv, c2, tpos)
            tposv = lax.fori_loop(0, nw, wave_b, zi)
            tpos = jnp.max(tposv)
            nt = (tpos + 15) // 16

            def count_ge(t):
                tv = zf + t

                def cb(v, c):
                    x = tval[pl.ds(v * 16, 16)]
                    return c + (x >= tv).astype(jnp.int32)
                return jnp.sum(lax.fori_loop(0, nt, cb, zi))

            def bisect(k):
                def it(_, lh):
                    lo, hi = lh
                    mid = 0.5 * (lo + hi)
                    big = count_ge(mid) >= k
                    return (jnp.where(big, mid, lo),
                            jnp.where(big, hi, mid))
                lo, _hi = lax.fori_loop(0, 30, it, (edge_t, mx + 1e-3))
                return lo
            t10 = bisect(10)
            t20 = bisect(20)
            t100 = bisect(100)
            t200 = bisect(200)
            t10v = zf + t10
            t20v = zf + t20
            t100v = zf + t100
            t200v = zf + t200
            itmp = 1.0 / _TEMP

            def zacc(v, z):
                x = tval[pl.ds(v * 16, 16)]
                return z + jnp.where(x >= t200v,
                                     jnp.exp((x - mx) * itmp), 0.0)
            zvec = lax.fori_loop(0, nt, zacc, zf)
            zinvv = recip16(zf + jnp.sum(zvec))

            def band_addr(x, lb):
                bidx = 3 - ((x >= t10v).astype(jnp.int32)
                            + (x >= t20v).astype(jnp.int32)
                            + (x >= t100v).astype(jnp.int32))
                return lane * _LP + bidx * _CLSP + lb

            def vote(v, c):
                x = tval[pl.ds(v * 16, 16)]
                lb = tlbl[pl.ds(v * 16, 16)]
                sel = x >= t200v
                w = jnp.exp((x - mx) * itmp) * zinvv
                plsc.addupdate_scatter(outsub, [band_addr(x, lb)], w,
                                       mask=sel)
                return c
            lax.fori_loop(0, nt, vote, 0)

            def mo(u, c):
                acc = zf
                for bb in range(4):
                    s = outsub[pl.ds(bb * _CLSP + u * 16, 16)]
                    for l in range(1, 16):
                        s = s + outsub[pl.ds(l * _LP
                                             + bb * _CLSP + u * 16, 16)]
                    acc = acc + s
                    outrow[pl.ds(bb * _CLSP + u * 16, 16)] = acc
                return c
            lax.fori_loop(0, _CLSP // 16, mo, 0)

            def unvote(v, c):
                x = tval[pl.ds(v * 16, 16)]
                lb = tlbl[pl.ds(v * 16, 16)]
                sel = x >= t200v
                plsc.store_scatter(outsub, [band_addr(x, lb)], zf,
                                   mask=sel)
                return c
            lax.fori_loop(0, nt, unvote, 0)

            co = [pltpu.async_copy(outrow.at[pl.ds(b * _CLSP, _CLSP)],
                                   dst.at[q], sem)
                  for b, dst in enumerate((o10, o20, o100, o200))]
            for c in co:
                c.wait()
            return _unused

        lax.fori_loop(0, _ROWS_PER_W, row_body, 0)

    return body(sims4, cm, labels4)


def kernel(features_rank, train_features, train_labels):
    sims = _similarity(features_rank, train_features)
    cm = _chunkmax128(sims)
    sims4 = sims.reshape(_Q * _NSC, 128)
    labels4 = jnp.pad(train_labels, (0, _NPAD - _N)).reshape(_NSC, 128)
    o10, o20, o100, o200 = _sc_vote(sims4, cm, labels4)
    return (o10[:, :_NCLS], o20[:, :_NCLS], o100[:, :_NCLS], o200[:, :_NCLS])
